# baseline (speedup 1.0000x reference)
import jax
import jax.numpy as jnp
from jax.experimental import pallas as pl

_BATCH_BLOCK = 16


def _broadcast_body(w_ref, o_ref):
    o_ref[...] = jnp.broadcast_to(w_ref[...], o_ref.shape)


def kernel(x, embed_weight):
    b, l, f = x.shape[0], x.shape[-2], embed_weight.shape[-1]
    table = embed_weight[:l]
    grid = (b // _BATCH_BLOCK,)
    return pl.pallas_call(
        _broadcast_body,
        grid=grid,
        in_specs=[pl.BlockSpec((l, f), lambda i: (0, 0))],
        out_specs=pl.BlockSpec((_BATCH_BLOCK, l, f), lambda i: (i, 0, 0)),
        out_shape=jax.ShapeDtypeStruct((b, l, f), embed_weight.dtype),
    )(table)
